# baseline (speedup 1.0000x reference)
import functools

import jax
import jax.numpy as jnp
from jax import lax
from jax.experimental import pallas as pl
from jax.experimental.pallas import tpu as pltpu
from jax.experimental.pallas import tpu_sc as plsc

_N, _D, _G, _DOUT = 10000, 512, 1024, 512
_R = 16
_NW = 32
_NPAD = 10240
_SPW = _G // _NW
_NCH = _NPAD // _R
_NBUF = 3


def _sc_pool(x_hbm, lp_hbm, idx_hbm, out_hbm,
             xb0, xb1, xb2, bbuf, lpb, accl, sem0, sem1, sem2):
    xbufs = (xb0, xb1, xb2)
    sems = (sem0, sem1, sem2)
    c = lax.axis_index("c")
    s = lax.axis_index("s")
    w = s * 2 + c
    glo = w * _SPW
    ghi = glo + _SPW

    pltpu.sync_copy(idx_hbm, bbuf.at[pl.ds(0, _N)])
    pltpu.sync_copy(lp_hbm, lpb)
    for t in range((_NPAD - _N) // _R):
        bbuf[pl.ds(_N + t * _R, _R)] = jnp.full((_R,), _G, jnp.int32)

    def zrow(r, carry):
        for cc in range(_D // 16):
            accl[r, pl.ds(cc * 16, 16)] = jnp.zeros((16,), jnp.float32)
        return carry
    lax.fori_loop(0, _SPW, zrow, 0)

    def lower_bound(bound):
        def step(_, lr):
            l, r = lr
            mid = (l + r) // 2
            first = bbuf[pl.ds(mid * _R, _R)][0]
            go = first < bound
            return jnp.where(go, mid + 1, l), jnp.where(go, r, mid)
        q, _ = lax.fori_loop(0, 10, step,
                             (jnp.int32(0), jnp.int32(_NCH)))
        qm = jnp.maximum(q - 1, 0)
        v = bbuf[pl.ds(qm * _R, _R)]
        cnt = jnp.int32(0)
        for r in range(_R):
            cnt = cnt + jnp.where(v[r] < bound, 1, 0).astype(jnp.int32)
        return jnp.where(q == 0, 0, qm * _R + cnt)
    lo = lower_bound(glo)
    hi = lower_bound(ghi)

    ch_lo = lo // _R
    ch_hi = (hi + _R - 1) // _R
    nch = ch_hi - ch_lo

    def xcopy(k, b):
        return pltpu.make_async_copy(
            x_hbm.at[pl.ds(k * _R, _R), :], xbufs[b], sems[b])

    for b in range(_NBUF):
        @pl.when(b < nch)
        def _():
            xcopy(ch_lo + b, b).start()

    def process(k, b):
        xcopy(k, b).wait()
        base = k * _R
        gv = bbuf[pl.ds(base, _R)]
        pv = jnp.exp(lpb[pl.ds(base, _R)])
        pv = jnp.where((gv >= glo) & (gv < ghi), pv, 0.0)
        gl = jnp.clip(gv - glo, 0, _SPW - 1)
        xb = xbufs[b]
        for r in range(_R):
            pr = pv[r]
            gr = gl[r]
            nb = _D // 16 // 8
            def lblk(cb):
                out = []
                for j in range(8):
                    sl = pl.ds((cb * 8 + j) * 16, 16)
                    out.append(xb[r, sl] * pr)
                return out
            def sblk(cb, vals):
                for j in range(8):
                    sl = pl.ds((cb * 8 + j) * 16, 16)
                    plsc.addupdate(accl.at[gr, sl], vals[j])
            prev = lblk(0)
            for cb in range(1, nb):
                cur = lblk(cb)
                sblk(cb - 1, prev)
                prev = cur
            sblk(nb - 1, prev)

        @pl.when(k + _NBUF < ch_hi)
        def _():
            xcopy(k + _NBUF, b).start()

    def round_(t, carry):
        k0 = ch_lo + _NBUF * t
        for b in range(_NBUF):
            @pl.when(k0 + b < ch_hi)
            def _(b=b):
                process(k0 + b, b)
        return carry
    lax.fori_loop(0, (nch + _NBUF - 1) // _NBUF, round_, 0)

    pltpu.sync_copy(accl, out_hbm.at[pl.ds(glo, _SPW), :])


def _mlp_body(pool_ref, lp_ref, w1_ref, b1_ref, w2_ref, b2_ref, out_ref):
    total = jnp.sum(jnp.exp(lp_ref[:]))
    pooled = pool_ref[:, :] / total
    h = jnp.maximum(
        jnp.dot(pooled, w1_ref[:, :], preferred_element_type=jnp.float32)
        + b1_ref[:, :], 0.0)
    out_ref[:, :] = jnp.dot(h, w2_ref[:, :],
                            preferred_element_type=jnp.float32) + b2_ref[:, :]


@functools.partial(
    pl.kernel,
    out_type=jax.ShapeDtypeStruct((_G, _D), jnp.float32),
    mesh=plsc.VectorSubcoreMesh(core_axis_name="c", subcore_axis_name="s"),
    scratch_types=[
        pltpu.VMEM((_R, _D), jnp.float32),
        pltpu.VMEM((_R, _D), jnp.float32),
        pltpu.VMEM((_R, _D), jnp.float32),
        pltpu.VMEM((_NPAD,), jnp.int32),
        pltpu.VMEM((_N,), jnp.float32),
        pltpu.VMEM((_SPW, _D), jnp.float32),
        pltpu.SemaphoreType.DMA,
        pltpu.SemaphoreType.DMA,
        pltpu.SemaphoreType.DMA,
    ],
)
def _sc_pool_call(x, lp, idx, out, *scratch):
    _sc_pool(x, lp, idx, out, *scratch)


def kernel(x, node_logprob, batch, y, W1, b1, W2, b2):
    pooled = _sc_pool_call(x, node_logprob, batch.astype(jnp.int32))
    pred = pl.pallas_call(
        _mlp_body,
        out_shape=jax.ShapeDtypeStruct((_G, _DOUT), jnp.float32),
    )(pooled, node_logprob, W1, b1.reshape(1, _D),
      W2, b2.reshape(1, _DOUT))
    return (pred, y)


# speedup vs baseline: 1.2351x; 1.2351x over previous
import functools

import jax
import jax.numpy as jnp
from jax import lax
from jax.experimental import pallas as pl
from jax.experimental.pallas import tpu as pltpu
from jax.experimental.pallas import tpu_sc as plsc

_N, _D, _G, _DOUT = 10000, 512, 1024, 512
_R = 16
_NW = 32
_NPAD = 10240
_SPW = _G // _NW
_NCH = _NPAD // _R
_NBUF = 2


def _sc_pool(x_hbm, lp_hbm, idx_hbm, out_hbm,
             xb0, xb1, bbuf, lpb, accl, sem0, sem1):
    xbufs = (xb0, xb1)
    sems = (sem0, sem1)
    c = lax.axis_index("c")
    s = lax.axis_index("s")
    w = s * 2 + c
    glo = w * _SPW
    ghi = glo + _SPW

    pltpu.sync_copy(idx_hbm, bbuf.at[pl.ds(0, _N)])
    pltpu.sync_copy(lp_hbm, lpb)
    for t in range((_NPAD - _N) // _R):
        bbuf[pl.ds(_N + t * _R, _R)] = jnp.full((_R,), _G, jnp.int32)

    def zrow(r, carry):
        for cc in range(_D // 16):
            accl[r, pl.ds(cc * 16, 16)] = jnp.zeros((16,), jnp.float32)
        return carry
    lax.fori_loop(0, _SPW, zrow, 0)

    def lower_bound(bound):
        def step(_, lr):
            l, r = lr
            mid = (l + r) // 2
            first = bbuf[pl.ds(mid * _R, _R)][0]
            go = first < bound
            return jnp.where(go, mid + 1, l), jnp.where(go, r, mid)
        q, _ = lax.fori_loop(0, 10, step,
                             (jnp.int32(0), jnp.int32(_NCH)))
        qm = jnp.maximum(q - 1, 0)
        v = bbuf[pl.ds(qm * _R, _R)]
        cnt = jnp.int32(0)
        for r in range(_R):
            cnt = cnt + jnp.where(v[r] < bound, 1, 0).astype(jnp.int32)
        return jnp.where(q == 0, 0, qm * _R + cnt)
    lo = lower_bound(glo)
    hi = lower_bound(ghi)

    def do16(base, xb, roff):
        gv = bbuf[pl.ds(base, _R)]
        pv = jnp.exp(lpb[pl.ds(base, _R)])
        pv = jnp.where((gv >= glo) & (gv < ghi), pv, 0.0)
        gl = jnp.clip(gv - glo, 0, _SPW - 1)
        for r in range(_R):
            pr = pv[r]
            gr = gl[r]
            row = roff + r
            nb = _D // 16 // 8
            def lblk(cb):
                out = []
                for j in range(8):
                    sl = pl.ds((cb * 8 + j) * 16, 16)
                    out.append(xb[row, sl] * pr)
                return out
            def sblk(cb, vals):
                for j in range(8):
                    sl = pl.ds((cb * 8 + j) * 16, 16)
                    plsc.addupdate(accl.at[gr, sl], vals[j])
            prev = lblk(0)
            for cb in range(1, nb):
                cur = lblk(cb)
                sblk(cb - 1, prev)
                prev = cur
            sblk(nb - 1, prev)

    _C64 = 4 * _R
    _NFULL = _N // _C64
    c_lo = lo // _C64
    c_hi = jnp.minimum((hi + _C64 - 1) // _C64, _NFULL)
    nch = c_hi - c_lo

    def xcopy(k, b):
        return pltpu.make_async_copy(
            x_hbm.at[pl.ds(k * _C64, _C64), :], xbufs[b], sems[b])

    for b in range(_NBUF):
        @pl.when(b < nch)
        def _(b=b):
            xcopy(c_lo + b, b).start()

    def process(k, b):
        xcopy(k, b).wait()
        xb = xbufs[b]

        def sub(i, carry):
            do16(k * _C64 + i * _R, xb, i * _R)
            return carry
        lax.fori_loop(0, _C64 // _R, sub, 0)

        @pl.when(k + _NBUF < c_hi)
        def _():
            xcopy(k + _NBUF, b).start()

    def round_(t, carry):
        k0 = c_lo + _NBUF * t
        for b in range(_NBUF):
            @pl.when(k0 + b < c_hi)
            def _(b=b):
                process(k0 + b, b)
        return carry
    lax.fori_loop(0, (nch + _NBUF - 1) // _NBUF, round_, 0)

    @pl.when(hi > _NFULL * _C64)
    def _():
        pltpu.sync_copy(x_hbm.at[pl.ds(_NFULL * _C64, _R), :],
                        xbufs[0].at[pl.ds(0, _R), :])
        do16(_NFULL * _C64, xbufs[0], 0)

    pltpu.sync_copy(accl, out_hbm.at[pl.ds(glo, _SPW), :])


def _mlp_body(pool_ref, lp_ref, w1_ref, b1_ref, w2_ref, b2_ref, out_ref):
    total = jnp.sum(jnp.exp(lp_ref[:]))
    pooled = pool_ref[:, :] / total
    h = jnp.maximum(
        jnp.dot(pooled, w1_ref[:, :], preferred_element_type=jnp.float32)
        + b1_ref[:, :], 0.0)
    out_ref[:, :] = jnp.dot(h, w2_ref[:, :],
                            preferred_element_type=jnp.float32) + b2_ref[:, :]


@functools.partial(
    pl.kernel,
    out_type=jax.ShapeDtypeStruct((_G, _D), jnp.float32),
    mesh=plsc.VectorSubcoreMesh(core_axis_name="c", subcore_axis_name="s"),
    scratch_types=[
        pltpu.VMEM((4 * _R, _D), jnp.float32),
        pltpu.VMEM((4 * _R, _D), jnp.float32),
        pltpu.VMEM((_NPAD,), jnp.int32),
        pltpu.VMEM((_N,), jnp.float32),
        pltpu.VMEM((_SPW, _D), jnp.float32),
        pltpu.SemaphoreType.DMA,
        pltpu.SemaphoreType.DMA,
    ],
)
def _sc_pool_call(x, lp, idx, out, *scratch):
    _sc_pool(x, lp, idx, out, *scratch)


def kernel(x, node_logprob, batch, y, W1, b1, W2, b2):
    pooled = _sc_pool_call(x, node_logprob, batch.astype(jnp.int32))
    pred = pl.pallas_call(
        _mlp_body,
        out_shape=jax.ShapeDtypeStruct((_G, _DOUT), jnp.float32),
    )(pooled, node_logprob, W1, b1.reshape(1, _D),
      W2, b2.reshape(1, _DOUT))
    return (pred, y)
